# baseline (speedup 1.0000x reference)
import functools

import jax
import jax.numpy as jnp
from jax import lax
from jax.experimental import pallas as pl
from jax.experimental.pallas import tpu as pltpu
from jax.experimental.pallas import tpu_sc as plsc


def _matvec_body(tt_ref, w_ref, b_ref, s_ref):
    s_ref[...] = (lax.dot_general(
        w_ref[...], tt_ref[...],
        dimension_numbers=(((1,), (0,)), ((), ())),
        preferred_element_type=jnp.float32,
    ) + b_ref[0, 0]).reshape(s_ref.shape)


def _scored_table(table, fc_w, fc_b):
    v, d = table.shape
    vb = 32768
    grid = -(-v // vb)
    s2 = pl.pallas_call(
        _matvec_body,
        grid=(grid,),
        in_specs=[
            pl.BlockSpec((d, vb), lambda i: (0, i)),
            pl.BlockSpec((1, d), lambda i: (0, 0)),
            pl.BlockSpec((1, 1), lambda i: (0, 0)),
        ],
        out_specs=pl.BlockSpec((vb,), lambda i: (i,)),
        out_shape=jax.ShapeDtypeStruct((v,), jnp.float32),
    )(table.T, fc_w, fc_b.reshape(1, 1))
    return s2


def _make_sc_pool(b: int, l: int):
    info = plsc.get_sparse_core_info()
    nc, ns = info.num_cores, info.num_subcores
    nw = nc * ns
    rows_w = b // nw
    n_idx = rows_w * l
    groups = rows_w // 16
    assert b % nw == 0 and rows_w % 16 == 0 and rows_w <= 128

    mesh = plsc.VectorSubcoreMesh(core_axis_name="c", subcore_axis_name="s")

    @functools.partial(
        pl.kernel,
        out_type=jax.ShapeDtypeStruct((b,), jnp.float32),
        mesh=mesh,
        scratch_types=[
            pltpu.VMEM((l, rows_w), jnp.int32),
            pltpu.VMEM((n_idx,), jnp.float32),
            pltpu.VMEM((rows_w,), jnp.float32),
            pltpu.SemaphoreType.DMA,
        ],
    )
    def pool(xt_hbm, s_hbm, out_hbm, idx_v, val_v, out_v, sem):
        wid = lax.axis_index("s") * nc + lax.axis_index("c")

        pltpu.sync_copy(xt_hbm.at[:, pl.ds(wid * rows_w, rows_w)], idx_v)

        def fire(c, _):
            pltpu.make_async_copy(
                s_hbm.at[idx_v.at[c]],
                val_v.at[pl.ds(c * rows_w, rows_w)],
                sem,
            ).start()
            return 0

        lax.fori_loop(0, l, fire, 0)

        def body(c, accs):
            pltpu.make_async_copy(
                s_hbm.at[idx_v.at[c]],
                val_v.at[pl.ds(c * rows_w, rows_w)],
                sem,
            ).wait()
            off = c * rows_w
            return tuple(
                accs[g] + val_v[pl.ds(off + g * 16, 16)] for g in range(groups)
            )

        accs = lax.fori_loop(
            0, l, body, tuple(jnp.zeros((16,), jnp.float32) for _ in range(groups))
        )

        inv_l = jnp.float32(1.0 / l)
        for g in range(groups):
            z = accs[g] * inv_l
            out_v[pl.ds(g * 16, 16)] = 1.0 / (1.0 + jnp.exp(-z))

        pltpu.sync_copy(out_v, out_hbm.at[pl.ds(wid * rows_w, rows_w)])

    return pool


def kernel(x, table, fc_w, fc_b):
    b, l = x.shape
    s = _scored_table(table, fc_w, fc_b)
    pool = _make_sc_pool(b, l)
    out = pool(x.T, s)
    return out.reshape(b, 1)
